# baseline (speedup 1.0000x reference)
import jax
import jax.numpy as jnp
from jax import lax
from jax.experimental import pallas as pl
from jax.experimental.pallas import tpu as pltpu

T = 2048
HIDDEN = 768
NUM_EXPERTS = 8
INTER = 256
GROUP = 4
TBLK = 256


def _router_combine(xb, gate_w, bias_row):
    logits = lax.dot_general(xb, gate_w, (((1,), (1,)), ((), ())),
                             preferred_element_type=jnp.float32)
    scores = jax.nn.sigmoid(logits)
    scores_c = scores + bias_row

    def top2sum(s4):
        a, b, c, d = (s4[:, 0], s4[:, 1], s4[:, 2], s4[:, 3])
        return jnp.maximum(
            jnp.maximum(jnp.maximum(a + b, a + c), jnp.maximum(a + d, b + c)),
            jnp.maximum(b + d, c + d))

    g0 = top2sum(scores_c[:, 0:GROUP])
    g1 = top2sum(scores_c[:, GROUP:2 * GROUP])
    sel0 = (g0 >= g1).astype(jnp.float32)[:, None]
    lane = lax.broadcasted_iota(jnp.int32, (TBLK, NUM_EXPERTS), 1)
    in_g0 = (lane < GROUP).astype(jnp.float32)
    maskf = sel0 * in_g0 + (1.0 - sel0) * (1.0 - in_g0)
    masked = scores_c * maskf - 1e9 * (1.0 - maskf)

    m = jnp.max(masked, axis=1, keepdims=True)
    eq = (masked == m).astype(jnp.float32)
    tri = (lax.broadcasted_iota(jnp.int32, (NUM_EXPERTS, NUM_EXPERTS), 0)
           < lax.broadcasted_iota(jnp.int32, (NUM_EXPERTS, NUM_EXPERTS), 1)
           ).astype(jnp.float32)
    prior = lax.dot_general(eq, tri, (((1,), (0,)), ((), ())),
                            preferred_element_type=jnp.float32)
    onehot = eq * (prior == 0.0).astype(jnp.float32)

    w = jnp.sum(onehot * scores, axis=1, keepdims=True)
    w = w / (w + 1e-20)
    return onehot * w


def _moe_body(x_ref, gate_w_ref, bias_ref, wgu_ref, wd_ref, sgu_ref, sd_ref,
              out_ref, wgu16, wd16, sgu16, sd16, h_scr):
    @pl.when(pl.program_id(0) == 0)
    def _cast_weights():
        wgu16[...] = wgu_ref[...].astype(jnp.bfloat16)
        wd16[...] = wd_ref[...].astype(jnp.bfloat16)
        sgu16[...] = sgu_ref[...].astype(jnp.bfloat16)
        sd16[...] = sd_ref[...].astype(jnp.bfloat16)

    xb = x_ref[...]
    combine = _router_combine(xb, gate_w_ref[...], bias_ref[...])

    xb16 = xb.astype(jnp.bfloat16)
    for e in range(NUM_EXPERTS):
        gu = lax.dot_general(xb16, wgu16[e], (((1,), (0,)), ((), ())),
                             preferred_element_type=jnp.float32)
        g = gu[:, :INTER]
        u = gu[:, INTER:]
        h = g * jax.nn.sigmoid(g) * u
        h_scr[:, e * INTER:(e + 1) * INTER] = (
            h * combine[:, e][:, None]).astype(jnp.bfloat16)
    acc = lax.dot_general(h_scr[...], wd16[...], (((1,), (0,)), ((), ())),
                          preferred_element_type=jnp.float32)

    sgu = lax.dot_general(xb16, sgu16[...], (((1,), (0,)), ((), ())),
                          preferred_element_type=jnp.float32)
    sg = sgu[:, :INTER]
    su = sgu[:, INTER:]
    sh = (sg * jax.nn.sigmoid(sg) * su).astype(jnp.bfloat16)
    shared = lax.dot_general(sh, sd16[...], (((1,), (0,)), ((), ())),
                             preferred_element_type=jnp.float32)
    out_ref[...] = acc + shared


def kernel(hidden_states, gate_w, correction_bias, w_gate_up, w_down,
           shared_gate_up, shared_down):
    bias_row = correction_bias.reshape(1, NUM_EXPERTS)
    wd_flat = w_down.reshape(NUM_EXPERTS * INTER, HIDDEN)
    grid = (T // TBLK,)
    return pl.pallas_call(
        _moe_body,
        grid=grid,
        in_specs=[
            pl.BlockSpec((TBLK, HIDDEN), lambda i: (i, 0)),
            pl.BlockSpec((NUM_EXPERTS, HIDDEN), lambda i: (0, 0)),
            pl.BlockSpec((1, NUM_EXPERTS), lambda i: (0, 0)),
            pl.BlockSpec((NUM_EXPERTS, HIDDEN, 2 * INTER), lambda i: (0, 0, 0)),
            pl.BlockSpec((NUM_EXPERTS * INTER, HIDDEN), lambda i: (0, 0)),
            pl.BlockSpec((HIDDEN, 2 * INTER), lambda i: (0, 0)),
            pl.BlockSpec((INTER, HIDDEN), lambda i: (0, 0)),
        ],
        out_specs=pl.BlockSpec((TBLK, HIDDEN), lambda i: (i, 0)),
        out_shape=jax.ShapeDtypeStruct((T, HIDDEN), jnp.float32),
        scratch_shapes=[
            pltpu.VMEM((NUM_EXPERTS, HIDDEN, 2 * INTER), jnp.bfloat16),
            pltpu.VMEM((NUM_EXPERTS * INTER, HIDDEN), jnp.bfloat16),
            pltpu.VMEM((HIDDEN, 2 * INTER), jnp.bfloat16),
            pltpu.VMEM((INTER, HIDDEN), jnp.bfloat16),
            pltpu.VMEM((TBLK, NUM_EXPERTS * INTER), jnp.bfloat16),
        ],
    )(hidden_states, gate_w, bias_row, w_gate_up, wd_flat,
      shared_gate_up, shared_down)
